# Initial kernel scaffold; baseline (speedup 1.0000x reference)
import jax
import jax.numpy as jnp
from jax.experimental import pallas as pl


def kernel(x, gate_w, w1, w2):
    raise NotImplementedError("write your pallas kernel here")


# speedup vs baseline: 1.7313x; 1.7313x over previous
import functools

import jax
import jax.numpy as jnp
from jax.experimental import pallas as pl
from jax.experimental.pallas import tpu as pltpu

E = 8
TOPK = 2
H = 1024
DFF = 1024
T = 2048
TB = 256


def _moe_body(x_ref, gw_ref, w1_ref, w2_ref, out_ref):
    e = pl.program_id(0)
    tb = pl.program_id(1)
    rows = pl.ds(tb * TB, TB)
    xb = x_ref[rows, :]

    logits = jax.lax.dot_general(
        xb, gw_ref[...], (((1,), (1,)), ((), ())),
        preferred_element_type=jnp.float32)
    m = jnp.max(logits, axis=1, keepdims=True)
    ex = jnp.exp(logits - m)
    probs = ex / jnp.sum(ex, axis=1, keepdims=True)
    lane = jax.lax.broadcasted_iota(jnp.int32, (TB, E), 1)
    m1 = jnp.max(probs, axis=1, keepdims=True)
    e1 = jnp.min(jnp.where(probs == m1, lane, E), axis=1, keepdims=True)
    pm = jnp.where(lane == e1, -1.0, probs)
    m2 = jnp.max(pm, axis=1, keepdims=True)
    e2 = jnp.min(jnp.where(pm == m2, lane, E), axis=1, keepdims=True)
    denom = m1 + m2
    wcol = jnp.where(e1 == e, m1, jnp.where(e2 == e, m2, 0.0)) / denom

    xb16 = xb.astype(jnp.bfloat16)
    xw16 = (xb * wcol).astype(jnp.bfloat16)
    w1e = w1_ref[0].astype(jnp.bfloat16)
    g = jax.lax.dot_general(
        xb16, w1e[:DFF, :], (((1,), (1,)), ((), ())),
        preferred_element_type=jnp.float32)
    u = jax.lax.dot_general(
        xw16, w1e[DFF:, :], (((1,), (1,)), ((), ())),
        preferred_element_type=jnp.float32)
    inter = (g * jax.nn.sigmoid(g) * u).astype(jnp.bfloat16)
    y = jax.lax.dot_general(
        inter, w2_ref[0].astype(jnp.bfloat16), (((1,), (1,)), ((), ())),
        preferred_element_type=jnp.float32)

    @pl.when(e == 0)
    def _init():
        out_ref[rows, :] = y

    @pl.when(e != 0)
    def _acc():
        out_ref[rows, :] += y


@jax.jit
def kernel(x, gate_w, w1, w2):
    out = pl.pallas_call(
        _moe_body,
        grid=(E, T // TB),
        in_specs=[
            pl.BlockSpec((T, H), lambda e, tb: (0, 0)),
            pl.BlockSpec((E, H), lambda e, tb: (0, 0)),
            pl.BlockSpec((1, 2 * DFF, H), lambda e, tb: (e, 0, 0)),
            pl.BlockSpec((1, H, DFF), lambda e, tb: (e, 0, 0)),
        ],
        out_specs=pl.BlockSpec((T, H), lambda e, tb: (0, 0)),
        out_shape=jax.ShapeDtypeStruct((T, H), jnp.float32),
        compiler_params=pltpu.CompilerParams(
            dimension_semantics=("arbitrary", "arbitrary"),
        ),
    )(x, gate_w, w1, w2)
    return out.reshape(T, 1, H)
